# baseline (speedup 1.0000x reference)
import jax
import jax.numpy as jnp
from jax.experimental import pallas as pl
from jax.experimental.pallas import tpu as pltpu


def _pos_body(row_ref, col_ref, out_ref):
    h, half = row_ref.shape
    w = col_ref.shape[0]
    hw = h * w
    lane = jax.lax.broadcasted_iota(jnp.int32, (w, hw), 1)
    src = jax.lax.broadcasted_iota(jnp.int32, (w, hw), 0)
    p = (lane % w == src).astype(jnp.float32)
    lane_h = jax.lax.broadcasted_iota(jnp.int32, (h, hw), 1)
    src_h = jax.lax.broadcasted_iota(jnp.int32, (h, hw), 0)
    q = (lane_h // w == src_h).astype(jnp.float32)
    xe = jax.lax.dot_general(
        col_ref[...], p, (((0,), (0,)), ((), ())),
        preferred_element_type=jnp.float32,
    )
    ye = jax.lax.dot_general(
        row_ref[...], q, (((0,), (0,)), ((), ())),
        preferred_element_type=jnp.float32,
    )
    out_ref[0, :half, :] = xe
    out_ref[0, half:, :] = ye


def kernel(x, row_w, col_w):
    n, dim, h, w = x.shape
    half = dim // 2
    out = pl.pallas_call(
        _pos_body,
        grid=(n,),
        in_specs=[
            pl.BlockSpec((h, half), lambda i: (0, 0)),
            pl.BlockSpec((w, half), lambda i: (0, 0)),
        ],
        out_specs=pl.BlockSpec((1, dim, h * w), lambda i: (i, 0, 0)),
        out_shape=jax.ShapeDtypeStruct((n, dim, h * w), jnp.float32),
        compiler_params=pltpu.CompilerParams(
            dimension_semantics=("parallel",),
        ),
    )(row_w[:h], col_w[:w])
    return out.reshape(n, dim, h, w)


# speedup vs baseline: 1.1129x; 1.1129x over previous
import jax
import jax.numpy as jnp
from jax.experimental import pallas as pl
from jax.experimental.pallas import tpu as pltpu


def kernel(x, row_w, col_w):
    n, dim, h, w = x.shape
    half = dim // 2
    hw = h * w

    def body(row_ref, col_ref, out_ref, buf, sem):
        lane = jax.lax.broadcasted_iota(jnp.int32, (w, hw), 1)
        src = jax.lax.broadcasted_iota(jnp.int32, (w, hw), 0)
        p = (lane % w == src).astype(jnp.float32)
        lane_h = jax.lax.broadcasted_iota(jnp.int32, (h, hw), 1)
        src_h = jax.lax.broadcasted_iota(jnp.int32, (h, hw), 0)
        q = (lane_h // w == src_h).astype(jnp.float32)
        xe = jax.lax.dot_general(
            col_ref[...], p, (((0,), (0,)), ((), ())),
            preferred_element_type=jnp.float32,
        )
        ye = jax.lax.dot_general(
            row_ref[...], q, (((0,), (0,)), ((), ())),
            preferred_element_type=jnp.float32,
        )
        for k in range(n):
            buf[k, 0:half, :] = xe
            buf[k, half:dim, :] = ye
        cp = pltpu.make_async_copy(buf, out_ref, sem)
        cp.start()
        cp.wait()

    out = pl.pallas_call(
        body,
        in_specs=[
            pl.BlockSpec(memory_space=pltpu.VMEM),
            pl.BlockSpec(memory_space=pltpu.VMEM),
        ],
        out_specs=pl.BlockSpec(memory_space=pl.ANY),
        out_shape=jax.ShapeDtypeStruct((n, dim, hw), jnp.float32),
        scratch_shapes=[
            pltpu.VMEM((n, dim, hw), jnp.float32),
            pltpu.SemaphoreType.DMA,
        ],
    )(row_w[:h], col_w[:w])
    return out.reshape(n, dim, h, w)


# speedup vs baseline: 1.1129x; 1.0001x over previous
import jax
import jax.numpy as jnp
from jax.experimental import pallas as pl
from jax.experimental.pallas import tpu as pltpu


def kernel(x, row_w, col_w):
    n, dim, h, w = x.shape
    half = dim // 2
    hw = h * w

    def body(row_ref, col_ref, out_ref, *rest):
        bufs, sem = rest[:-1], rest[-1]
        lane = jax.lax.broadcasted_iota(jnp.int32, (w, hw), 1)
        src = jax.lax.broadcasted_iota(jnp.int32, (w, hw), 0)
        p = (lane % w == src).astype(jnp.float32)
        lane_h = jax.lax.broadcasted_iota(jnp.int32, (h, hw), 1)
        src_h = jax.lax.broadcasted_iota(jnp.int32, (h, hw), 0)
        q = (lane_h // w == src_h).astype(jnp.float32)
        xe = jax.lax.dot_general(
            col_ref[...], p, (((0,), (0,)), ((), ())),
            preferred_element_type=jnp.float32,
        )
        ye = jax.lax.dot_general(
            row_ref[...], q, (((0,), (0,)), ((), ())),
            preferred_element_type=jnp.float32,
        )
        for k in range(n):
            bufs[k][0:half, :] = xe
            bufs[k][half:dim, :] = ye
            pltpu.make_async_copy(bufs[k], out_ref.at[k], sem.at[k]).start()
        for k in range(n):
            pltpu.make_async_copy(bufs[k], out_ref.at[k], sem.at[k]).wait()

    out = pl.pallas_call(
        body,
        in_specs=[
            pl.BlockSpec(memory_space=pltpu.VMEM),
            pl.BlockSpec(memory_space=pltpu.VMEM),
        ],
        out_specs=pl.BlockSpec(memory_space=pl.ANY),
        out_shape=jax.ShapeDtypeStruct((n, dim, hw), jnp.float32),
        scratch_shapes=(
            [pltpu.VMEM((dim, hw), jnp.float32) for _ in range(n)]
            + [pltpu.SemaphoreType.DMA((n,))]
        ),
    )(row_w[:h], col_w[:w])
    return out.reshape(n, dim, h, w)
